# baseline (speedup 1.0000x reference)
import functools

import jax
import jax.numpy as jnp
from jax import lax
from jax.experimental import pallas as pl
from jax.experimental.pallas import tpu as pltpu
from jax.experimental.pallas import tpu_sc as plsc

N = 100000
D = 576
S = 4
D_K = 64
S1, S2, S3 = 256, 512, 576
K = 4096
D_B = 64
D_A = 64
R = 4

N_BLK = 2048

CW = 4096
NFULL = N // CW
TAIL_LO = (N // 128) * 128
CW_LAST = TAIL_LO - NFULL * CW
RG = D // 8


def _keys_body(w_ref, xt_ref, out_ref):
    out_ref[...] = jnp.dot(w_ref[...], xt_ref[...],
                           preferred_element_type=jnp.float32)


def _tail_body(xt_ref, out_ref):
    out_ref[...] = xt_ref[...]


def _extract_tail(xt):
    return pl.pallas_call(
        _tail_body,
        grid=(1,),
        in_specs=[pl.BlockSpec((D, 128), lambda i: (0, TAIL_LO // 128))],
        out_specs=pl.BlockSpec((D, 128), lambda i: (0, 0)),
        out_shape=jax.ShapeDtypeStruct((D, 128), jnp.float32),
    )(xt)


def _compute_keys(w2, xt):
    return pl.pallas_call(
        _keys_body,
        grid=(pl.cdiv(N, N_BLK),),
        in_specs=[
            pl.BlockSpec((S * D_K, D), lambda i: (0, 0)),
            pl.BlockSpec((D, N_BLK), lambda i: (0, i)),
        ],
        out_specs=pl.BlockSpec((S * D_K, N_BLK), lambda i: (0, i)),
        out_shape=jax.ShapeDtypeStruct((S * D_K, N), jnp.float32),
    )(w2, xt)


@functools.lru_cache(maxsize=None)
def _make_col_gather():
    info = plsc.get_sparse_core_info()
    nw = info.num_cores * info.num_subcores
    mesh = plsc.VectorSubcoreMesh(core_axis_name="c", subcore_axis_name="s")

    @functools.partial(
        pl.kernel,
        mesh=mesh,
        compiler_params=pltpu.CompilerParams(needs_layout_passes=False),
        out_type=jax.ShapeDtypeStruct((D, K), jnp.float32),
        scratch_types=[
            pltpu.VMEM((K + 16,), jnp.int32),
            pltpu.VMEM((K + 16,), jnp.int32),
            pltpu.VMEM((48,), jnp.int32),
            pltpu.VMEM((8, CW), jnp.float32),
            pltpu.VMEM((8, CW), jnp.float32),
            pltpu.VMEM((8, K), jnp.float32),
            pltpu.SemaphoreType.DMA,
            pltpu.SemaphoreType.DMA,
        ],
    )
    def gather_k(xt_hbm, tail_hbm, sidx_hbm, perm_hbm, bounds_hbm, out_hbm,
                 sidx_v, perm_v, bounds_v, buf_a, buf_b, obuf, sem_a, sem_b):
        w = lax.axis_index("s") * info.num_cores + lax.axis_index("c")
        pltpu.sync_copy(sidx_hbm, sidx_v)
        pltpu.sync_copy(perm_hbm, perm_v)
        pltpu.sync_copy(bounds_hbm, bounds_v)
        lane = lax.iota(jnp.int32, 16)

        def gather_seg(buf, lo_col, ent_lo, ent_hi):
            def g_body(j, carry):
                ent = ent_lo + j * 16
                nlv = sidx_v[pl.ds(ent, 16)] - lo_col
                kv = perm_v[pl.ds(ent, 16)]
                m = (ent + lane) < ent_hi

                for i in range(8):
                    row = jnp.full((16,), i, jnp.int32)
                    vals = plsc.load_gather(buf, [row, nlv], mask=m)
                    plsc.store_scatter(obuf, [row, kv], vals, mask=m)
                return carry

            lax.fori_loop(0, (ent_hi - ent_lo + 15) // 16, g_body,
                          jnp.int32(0))

        def bound_at(c):
            return bounds_v[pl.ds(c, 16)][0]

        def src(rg, c):
            return xt_hbm.at[pl.ds(rg * 8, 8), pl.ds(c * CW, CW)]

        def per_rg(rg):
            pltpu.async_copy(src(rg, 0), buf_a, sem_a)

            def pair_body(t, carry):
                c0 = 2 * t
                pltpu.async_copy(src(rg, c0 + 1), buf_b, sem_b)
                pltpu.make_async_copy(src(rg, c0), buf_a, sem_a).wait()
                gather_seg(buf_a, c0 * CW, bound_at(c0), bound_at(c0 + 1))

                @pl.when(t < NFULL // 2 - 1)
                def _():
                    pltpu.async_copy(src(rg, c0 + 2), buf_a, sem_a)

                pltpu.make_async_copy(src(rg, c0 + 1), buf_b, sem_b).wait()
                gather_seg(buf_b, (c0 + 1) * CW, bound_at(c0 + 1),
                           bound_at(c0 + 2))
                return carry

            lax.fori_loop(0, NFULL // 2, pair_body, jnp.int32(0))
            pltpu.sync_copy(
                xt_hbm.at[pl.ds(rg * 8, 8), pl.ds(NFULL * CW, CW_LAST)],
                buf_a.at[:, :CW_LAST])
            gather_seg(buf_a, NFULL * CW, bound_at(NFULL),
                       bound_at(NFULL + 1))
            pltpu.sync_copy(tail_hbm.at[pl.ds(rg * 8, 8), :],
                            buf_a.at[:, :128])
            gather_seg(buf_a, TAIL_LO, bound_at(NFULL + 1), jnp.int32(K))
            pltpu.sync_copy(obuf, out_hbm.at[pl.ds(rg * 8, 8), :])

        def per_slot(t, carry):
            rg = w + t * nw

            @pl.when(rg < RG)
            def _():
                per_rg(rg)

            return carry

        lax.fori_loop(0, (RG + nw - 1) // nw, per_slot, jnp.int32(0))

    return gather_k


def kernel(vectors, key_proj, indices):
    xt = vectors.T
    w2 = jnp.transpose(key_proj, (0, 2, 1)).reshape(S * D_K, D)
    out2 = _compute_keys(w2, xt)
    keys = out2.reshape(S, D_K, N).transpose(0, 2, 1)
    xtail = _extract_tail(xt)
    sidx, perm = lax.sort([indices, lax.iota(jnp.int32, K)], num_keys=1)
    edges = jnp.concatenate([
        jnp.arange(NFULL + 1, dtype=jnp.int32) * CW,
        jnp.array([TAIL_LO], dtype=jnp.int32),
    ])
    bounds = jnp.searchsorted(sidx, edges).astype(jnp.int32)
    bounds = jnp.concatenate(
        [bounds, jnp.full((48 - NFULL - 2,), K, jnp.int32)])
    pad16 = jnp.zeros((16,), jnp.int32)
    vecsT = _make_col_gather()(
        xt, xtail,
        jnp.concatenate([sidx, pad16]),
        jnp.concatenate([perm, pad16]),
        bounds,
    )
    U = vecsT[:S1].reshape(D_B, R, K).transpose(2, 0, 1)
    V = vecsT[S1:S2].reshape(R, D_A, K).transpose(2, 0, 1)
    b = vecsT[S2:S3].transpose(1, 0)
    return keys, U, V, b


# speedup vs baseline: 1.1155x; 1.1155x over previous
import functools

import jax
import jax.numpy as jnp
from jax import lax
from jax.experimental import pallas as pl
from jax.experimental.pallas import tpu as pltpu
from jax.experimental.pallas import tpu_sc as plsc

N = 100000
D = 576
DPAD = 640
S = 4
D_K = 64
S1, S2, S3 = 256, 512, 576
K = 4096
D_B = 64
D_A = 64
R = 4

N_BLK = 2048


def _keys_body(w_ref, xt_ref, out_ref, vpad_ref):
    xt = xt_ref[...]
    out_ref[...] = jnp.dot(w_ref[...], xt, preferred_element_type=jnp.float32)
    vpad_ref[:, :D] = xt.T


def _compute(w2, xt):
    return pl.pallas_call(
        _keys_body,
        grid=(pl.cdiv(N, N_BLK),),
        in_specs=[
            pl.BlockSpec((S * D_K, D), lambda i: (0, 0)),
            pl.BlockSpec((D, N_BLK), lambda i: (0, i)),
        ],
        out_specs=[
            pl.BlockSpec((S * D_K, N_BLK), lambda i: (0, i)),
            pl.BlockSpec((N_BLK, DPAD), lambda i: (i, 0)),
        ],
        out_shape=[
            jax.ShapeDtypeStruct((S * D_K, N), jnp.float32),
            jax.ShapeDtypeStruct((N, DPAD), jnp.float32),
        ],
    )(w2, xt)


@functools.lru_cache(maxsize=None)
def _make_gather():
    info = plsc.get_sparse_core_info()
    nw = info.num_cores * info.num_subcores
    b_per_w = K // nw
    mesh = plsc.VectorSubcoreMesh(core_axis_name="c", subcore_axis_name="s")

    @functools.partial(
        pl.kernel,
        mesh=mesh,
        out_type=jax.ShapeDtypeStruct((K, DPAD), jnp.float32),
        scratch_types=[
            pltpu.VMEM((b_per_w,), jnp.int32),
            pltpu.VMEM((b_per_w, DPAD), jnp.float32),
            pltpu.SemaphoreType.DMA,
        ],
    )
    def gather_k(table_hbm, idx_hbm, out_hbm, idx_v, rows_v, sem):
        wid = lax.axis_index("s") * info.num_cores + lax.axis_index("c")
        base = wid * b_per_w
        pltpu.sync_copy(idx_hbm.at[pl.ds(base, b_per_w)], idx_v)
        pltpu.async_copy(table_hbm.at[idx_v], rows_v, sem).wait()
        pltpu.sync_copy(rows_v, out_hbm.at[pl.ds(base, b_per_w)])

    return gather_k


def kernel(vectors, key_proj, indices):
    xt = vectors.T
    w2 = jnp.transpose(key_proj, (0, 2, 1)).reshape(S * D_K, D)
    out2, vpad = _compute(w2, xt)
    keys = out2.reshape(S, D_K, N).transpose(0, 2, 1)
    vecs = _make_gather()(vpad, indices)
    U = vecs[:, :S1].reshape(-1, D_B, R)
    V = vecs[:, S1:S2].reshape(-1, R, D_A)
    b = vecs[:, S2:S3]
    return keys, U, V, b
